# baseline (speedup 1.0000x reference)
import functools

import jax
import jax.numpy as jnp
from jax import lax
from jax.experimental import pallas as pl
from jax.experimental.pallas import tpu as pltpu
from jax.experimental.pallas import tpu_sc as plsc

DIM = 768
INTER = 384
E = 64
CAP = 160
N_TOK = 4096
NROWS = E * CAP + CAP
N_PAIR = 2 * N_TOK

TB = 256
NB = N_TOK // TB

NC = 2
NS = 16
NW = NC * NS
CHUNK = 32
PAIR_PER_W = N_PAIR // NW
CHUNKS_PER_W = PAIR_PER_W // CHUNK


def _router_body(x_ref, gw_ref, ds_ref, dg_ref, f_ref, sv_ref, carry_ref):
    b = pl.program_id(0)

    @pl.when(b == 0)
    def _():
        carry_ref[0:1, :] = jnp.zeros((1, E), jnp.float32)

    xb = x_ref[...]
    logits = lax.dot_general(xb, gw_ref[...],
                             (((1,), (1,)), ((), ())),
                             preferred_element_type=jnp.float32)
    m = jnp.max(logits, axis=1, keepdims=True)
    p = jnp.exp(logits - m)
    scores = p / jnp.sum(p, axis=1, keepdims=True)

    lane = lax.broadcasted_iota(jnp.int32, (TB, E), 1)
    s1 = jnp.max(scores, axis=1, keepdims=True)
    e0 = jnp.min(jnp.where(scores == s1, lane, E), axis=1, keepdims=True)
    sc2 = jnp.where(lane == e0, -1.0, scores)
    s2 = jnp.max(sc2, axis=1, keepdims=True)
    e1 = jnp.min(jnp.where(sc2 == s2, lane, E), axis=1, keepdims=True)
    denom = s1 + s2 + 1e-9
    w0 = s1 / denom
    w1 = s2 / denom

    oh0 = (lane == e0).astype(jnp.float32)
    oh1 = (lane == e1).astype(jnp.float32)
    cb = oh0 + oh1

    r = lax.broadcasted_iota(jnp.int32, (TB, TB), 0)
    c = lax.broadcasted_iota(jnp.int32, (TB, TB), 1)
    tril = (r > c).astype(jnp.float32)
    carry = carry_ref[0:1, :]
    base = lax.dot_general(tril, cb, (((1,), (0,)), ((), ())),
                           preferred_element_type=jnp.float32) + carry
    pos0 = jnp.sum(base * oh0, axis=1, keepdims=True).astype(jnp.int32)
    pos1 = jnp.sum(base * oh1, axis=1, keepdims=True).astype(jnp.int32)
    new_carry = carry + jnp.sum(cb, axis=0, keepdims=True)
    carry_ref[0:1, :] = new_carry

    v0 = pos0 < CAP
    v1 = pos1 < CAP
    d0 = e0 * CAP + pos0
    d1 = e1 * CAP + pos1
    tglob = b * TB + lax.broadcasted_iota(jnp.int32, (TB, 1), 0)
    dump = E * CAP + lax.rem(tglob, 32)
    ds_ref[...] = jnp.concatenate(
        [jnp.where(v0, d0, dump).reshape(1, 1, TB),
         jnp.where(v1, d1, dump).reshape(1, 1, TB)], axis=1)
    dg_ref[...] = jnp.concatenate(
        [jnp.where(v0, d0, 0).reshape(1, 1, TB),
         jnp.where(v1, d1, 0).reshape(1, 1, TB)], axis=1)
    f_ref[...] = jnp.concatenate(
        [jnp.where(v0, w0, 0.0), jnp.where(v1, w1, 0.0)], axis=1)

    @pl.when(b == NB - 1)
    def _():
        counts = new_carry.astype(jnp.int32).reshape(E, 1, 1)
        sl = lax.broadcasted_iota(jnp.int32, (E, CAP, 1), 1)
        sv_ref[...] = (sl < counts).astype(jnp.float32)


def _router(x, gate_w):
    return pl.pallas_call(
        _router_body,
        grid=(NB,),
        in_specs=[
            pl.BlockSpec((TB, DIM), lambda b: (b, 0)),
            pl.BlockSpec((E, DIM), lambda b: (0, 0)),
        ],
        out_specs=[
            pl.BlockSpec((1, 2, TB), lambda b: (b, 0, 0)),
            pl.BlockSpec((1, 2, TB), lambda b: (b, 0, 0)),
            pl.BlockSpec((TB, 2), lambda b: (b, 0)),
            pl.BlockSpec((E, CAP, 1), lambda b: (0, 0, 0)),
        ],
        out_shape=[
            jax.ShapeDtypeStruct((NB, 2, TB), jnp.int32),
            jax.ShapeDtypeStruct((NB, 2, TB), jnp.int32),
            jax.ShapeDtypeStruct((N_TOK, 2), jnp.float32),
            jax.ShapeDtypeStruct((E, CAP, 1), jnp.float32),
        ],
        scratch_shapes=[pltpu.VMEM((8, E), jnp.float32)],
    )(x, gate_w)


def _sc_dispatch(x, dests3):
    mesh = plsc.VectorSubcoreMesh(core_axis_name="c", subcore_axis_name="s")

    @functools.partial(
        pl.kernel, mesh=mesh,
        out_type=jax.ShapeDtypeStruct((NROWS, DIM), jnp.float32),
        scratch_types=[
            pltpu.VMEM((CHUNKS_PER_W, CHUNK), jnp.int32),
            pltpu.VMEM((CHUNK, DIM), jnp.float32),
            pltpu.SemaphoreType.DMA,
        ],
    )
    def k(x_hbm, d_hbm, xg_hbm, idx_v, rows_v, sem):
        wid = lax.axis_index("s") * NC + lax.axis_index("c")
        base_tok = (wid // 2) * TB
        pltpu.sync_copy(d_hbm.at[wid], idx_v)
        for c in range(CHUNKS_PER_W):
            pltpu.sync_copy(x_hbm.at[pl.ds(base_tok + c * CHUNK, CHUNK)],
                            rows_v)
            pltpu.async_copy(rows_v, xg_hbm.at[idx_v.at[c]], sem).wait()

    return k(x, dests3)


def _sc_gather(out_all, destg3):
    mesh = plsc.VectorSubcoreMesh(core_axis_name="c", subcore_axis_name="s")

    @functools.partial(
        pl.kernel, mesh=mesh,
        out_type=jax.ShapeDtypeStruct((N_PAIR, DIM), jnp.float32),
        scratch_types=[
            pltpu.VMEM((CHUNKS_PER_W, CHUNK), jnp.int32),
            pltpu.VMEM((CHUNK, DIM), jnp.float32),
            pltpu.SemaphoreType.DMA,
        ],
    )
    def k(src_hbm, d_hbm, yg_hbm, idx_v, rows_v, sem):
        wid = lax.axis_index("s") * NC + lax.axis_index("c")
        pltpu.sync_copy(d_hbm.at[wid], idx_v)
        for c in range(CHUNKS_PER_W):
            pltpu.async_copy(src_hbm.at[idx_v.at[c]], rows_v, sem).wait()
            pltpu.sync_copy(rows_v,
                            yg_hbm.at[pl.ds(wid * PAIR_PER_W + c * CHUNK, CHUNK)])

    return k(out_all, destg3)


def _ffn_body(xg_ref, w1_ref, w3_ref, w2_ref, sv_ref, out_ref):
    xb = xg_ref[...]
    a = lax.dot_general(xb, w1_ref[0], (((1,), (1,)), ((), ())),
                        preferred_element_type=jnp.float32)
    g = lax.dot_general(xb, w3_ref[0], (((1,), (1,)), ((), ())),
                        preferred_element_type=jnp.float32)
    h = a * jax.nn.sigmoid(a) * g
    out = lax.dot_general(h, w2_ref[0], (((1,), (1,)), ((), ())),
                          preferred_element_type=jnp.float32)
    sv = sv_ref[0]
    out_ref[...] = jnp.where(sv > 0.5, out, 0.0)


def _ffn(xg, w1, w2, w3, slotvalid):
    return pl.pallas_call(
        _ffn_body,
        grid=(E,),
        in_specs=[
            pl.BlockSpec((CAP, DIM), lambda e: (e, 0)),
            pl.BlockSpec((1, INTER, DIM), lambda e: (e, 0, 0)),
            pl.BlockSpec((1, INTER, DIM), lambda e: (e, 0, 0)),
            pl.BlockSpec((1, DIM, INTER), lambda e: (e, 0, 0)),
            pl.BlockSpec((1, CAP, 1), lambda e: (e, 0, 0)),
        ],
        out_specs=pl.BlockSpec((CAP, DIM), lambda e: (e, 0)),
        out_shape=jax.ShapeDtypeStruct((E * CAP, DIM), jnp.float32),
    )(xg[:E * CAP], w1, w3, w2, slotvalid)


def _shared_body(x_ref, sw1_ref, sw3_ref, sw2_ref, z_ref):
    xb = x_ref[...]
    a = lax.dot_general(xb, sw1_ref[...], (((1,), (1,)), ((), ())),
                        preferred_element_type=jnp.float32)
    g = lax.dot_general(xb, sw3_ref[...], (((1,), (1,)), ((), ())),
                        preferred_element_type=jnp.float32)
    h = a * jax.nn.sigmoid(a) * g
    z_ref[...] = lax.dot_general(h, sw2_ref[...], (((1,), (1,)), ((), ())),
                                 preferred_element_type=jnp.float32)


def _shared(x, sw1, sw2, sw3):
    return pl.pallas_call(
        _shared_body,
        grid=(NB,),
        in_specs=[
            pl.BlockSpec((TB, DIM), lambda b: (b, 0)),
            pl.BlockSpec((DIM, DIM), lambda b: (0, 0)),
            pl.BlockSpec((DIM, DIM), lambda b: (0, 0)),
            pl.BlockSpec((DIM, DIM), lambda b: (0, 0)),
        ],
        out_specs=pl.BlockSpec((TB, DIM), lambda b: (b, 0)),
        out_shape=jax.ShapeDtypeStruct((N_TOK, DIM), jnp.float32),
    )(x, sw1, sw3, sw2)


def _combine_body(z_ref, yg_ref, f_ref, y_ref):
    f0 = f_ref[:, 0:1]
    f1 = f_ref[:, 1:2]
    y_ref[...] = z_ref[...] + f0 * yg_ref[0, 0] + f1 * yg_ref[0, 1]


def _combine(z, yg, f):
    return pl.pallas_call(
        _combine_body,
        grid=(NB,),
        in_specs=[
            pl.BlockSpec((TB, DIM), lambda b: (b, 0)),
            pl.BlockSpec((1, 2, TB, DIM), lambda b: (b, 0, 0, 0)),
            pl.BlockSpec((TB, 2), lambda b: (b, 0)),
        ],
        out_specs=pl.BlockSpec((TB, DIM), lambda b: (b, 0)),
        out_shape=jax.ShapeDtypeStruct((N_TOK, DIM), jnp.float32),
    )(z, yg, f)


def kernel(x, gate_w, w1, w2, w3, sw1, sw2, sw3):
    dest_s, dest_g, f, slotvalid = _router(x, gate_w)
    dests3 = dest_s.reshape(NW, CHUNKS_PER_W, CHUNK)
    destg3 = dest_g.reshape(NW, CHUNKS_PER_W, CHUNK)
    xg = _sc_dispatch(x, dests3)
    out_all = _ffn(xg, w1, w2, w3, slotvalid)
    yg = _sc_gather(out_all, destg3).reshape(NB, 2, TB, DIM)
    z = _shared(x, sw1, sw2, sw3)
    return _combine(z, yg, f)


# speedup vs baseline: 1.0461x; 1.0461x over previous
import functools

import jax
import jax.numpy as jnp
from jax import lax
from jax.experimental import pallas as pl
from jax.experimental.pallas import tpu as pltpu
from jax.experimental.pallas import tpu_sc as plsc

DIM = 768
INTER = 384
E = 64
CAP = 160
N_TOK = 4096
NROWS = E * CAP + CAP
N_PAIR = 2 * N_TOK

TB = 256
NB = N_TOK // TB

NC = 2
NS = 16
NW = NC * NS
CHUNK = 32
PAIR_PER_W = N_PAIR // NW
CHUNKS_PER_W = PAIR_PER_W // CHUNK


def _router_body(x_ref, gw_ref, ds_ref, dg_ref, f_ref, sv_ref, carry_ref):
    b = pl.program_id(0)

    @pl.when(b == 0)
    def _():
        carry_ref[0:1, :] = jnp.zeros((1, E), jnp.float32)

    xb = x_ref[...]
    logits = lax.dot_general(xb, gw_ref[...],
                             (((1,), (1,)), ((), ())),
                             preferred_element_type=jnp.float32)
    m = jnp.max(logits, axis=1, keepdims=True)
    p = jnp.exp(logits - m)
    scores = p / jnp.sum(p, axis=1, keepdims=True)

    lane = lax.broadcasted_iota(jnp.int32, (TB, E), 1)
    s1 = jnp.max(scores, axis=1, keepdims=True)
    e0 = jnp.min(jnp.where(scores == s1, lane, E), axis=1, keepdims=True)
    sc2 = jnp.where(lane == e0, -1.0, scores)
    s2 = jnp.max(sc2, axis=1, keepdims=True)
    e1 = jnp.min(jnp.where(sc2 == s2, lane, E), axis=1, keepdims=True)
    denom = s1 + s2 + 1e-9
    w0 = s1 / denom
    w1 = s2 / denom

    oh0 = (lane == e0).astype(jnp.float32)
    oh1 = (lane == e1).astype(jnp.float32)
    cb = oh0 + oh1

    r = lax.broadcasted_iota(jnp.int32, (TB, TB), 0)
    c = lax.broadcasted_iota(jnp.int32, (TB, TB), 1)
    tril = (r > c).astype(jnp.float32)
    carry = carry_ref[0:1, :]
    base = lax.dot_general(tril, cb, (((1,), (0,)), ((), ())),
                           preferred_element_type=jnp.float32) + carry
    pos0 = jnp.sum(base * oh0, axis=1, keepdims=True).astype(jnp.int32)
    pos1 = jnp.sum(base * oh1, axis=1, keepdims=True).astype(jnp.int32)
    new_carry = carry + jnp.sum(cb, axis=0, keepdims=True)
    carry_ref[0:1, :] = new_carry

    v0 = pos0 < CAP
    v1 = pos1 < CAP
    d0 = e0 * CAP + pos0
    d1 = e1 * CAP + pos1
    tglob = b * TB + lax.broadcasted_iota(jnp.int32, (TB, 1), 0)
    dump = E * CAP + lax.rem(tglob, 32)
    s0 = jnp.where(v0, d0, dump)
    s1 = jnp.where(v1, d1, dump)
    ds_ref[...] = jnp.concatenate(
        [s0.reshape(1, 2, 4, 1, CHUNK), s1.reshape(1, 2, 4, 1, CHUNK)],
        axis=3).reshape(1, 2, 8, CHUNK)
    dg_ref[...] = jnp.concatenate(
        [jnp.where(v0, d0, 0).reshape(1, 1, TB),
         jnp.where(v1, d1, 0).reshape(1, 1, TB)], axis=1)
    f_ref[...] = jnp.concatenate(
        [jnp.where(v0, w0, 0.0), jnp.where(v1, w1, 0.0)], axis=1)

    @pl.when(b == NB - 1)
    def _():
        counts = new_carry.astype(jnp.int32).reshape(E, 1, 1)
        sl = lax.broadcasted_iota(jnp.int32, (E, CAP, 1), 1)
        sv_ref[...] = (sl < counts).astype(jnp.float32)


def _router(x, gate_w):
    return pl.pallas_call(
        _router_body,
        grid=(NB,),
        in_specs=[
            pl.BlockSpec((TB, DIM), lambda b: (b, 0)),
            pl.BlockSpec((E, DIM), lambda b: (0, 0)),
        ],
        out_specs=[
            pl.BlockSpec((1, 2, 8, CHUNK), lambda b: (b, 0, 0, 0)),
            pl.BlockSpec((1, 2, TB), lambda b: (b, 0, 0)),
            pl.BlockSpec((TB, 2), lambda b: (b, 0)),
            pl.BlockSpec((E, CAP, 1), lambda b: (0, 0, 0)),
        ],
        out_shape=[
            jax.ShapeDtypeStruct((NB, 2, 8, CHUNK), jnp.int32),
            jax.ShapeDtypeStruct((NB, 2, TB), jnp.int32),
            jax.ShapeDtypeStruct((N_TOK, 2), jnp.float32),
            jax.ShapeDtypeStruct((E, CAP, 1), jnp.float32),
        ],
        scratch_shapes=[pltpu.VMEM((8, E), jnp.float32)],
    )(x, gate_w)


def _sc_dispatch(x, dests3):
    mesh = plsc.VectorSubcoreMesh(core_axis_name="c", subcore_axis_name="s")

    nck = 4

    @functools.partial(
        pl.kernel, mesh=mesh,
        out_type=jax.ShapeDtypeStruct((NROWS, DIM), jnp.float32),
        scratch_types=[
            pltpu.VMEM((2 * nck, CHUNK), jnp.int32),
            pltpu.VMEM((2, CHUNK, DIM), jnp.float32),
            pltpu.SemaphoreType.DMA,
            pltpu.SemaphoreType.DMA,
            pltpu.SemaphoreType.DMA,
            pltpu.SemaphoreType.DMA,
        ],
    )
    def k(x_hbm, d_hbm, xg_hbm, idx_v, rows_v, r0, r1, s0, s1):
        wid = lax.axis_index("s") * NC + lax.axis_index("c")
        base_tok = wid * (nck * CHUNK)
        pltpu.sync_copy(d_hbm.at[wid], idx_v)
        rsem = [r0, r1]
        ssem = [s0, s1]
        rd = [None, None]
        wr = [None, None]
        for c in range(nck):
            p = c % 2
            if c >= 2:
                wr[p][0].wait()
                wr[p][1].wait()
            rd[p] = pltpu.async_copy(
                x_hbm.at[pl.ds(base_tok + c * CHUNK, CHUNK)],
                rows_v.at[p], rsem[p])
            if c >= 1:
                q = (c - 1) % 2
                rd[q].wait()
                wr[q] = (
                    pltpu.async_copy(rows_v.at[q],
                                     xg_hbm.at[idx_v.at[2 * (c - 1)]], ssem[q]),
                    pltpu.async_copy(rows_v.at[q],
                                     xg_hbm.at[idx_v.at[2 * (c - 1) + 1]],
                                     ssem[q]),
                )
        q = (nck - 1) % 2
        rd[q].wait()
        wr[q] = (
            pltpu.async_copy(rows_v.at[q],
                             xg_hbm.at[idx_v.at[2 * (nck - 1)]], ssem[q]),
            pltpu.async_copy(rows_v.at[q],
                             xg_hbm.at[idx_v.at[2 * (nck - 1) + 1]], ssem[q]),
        )
        for p in range(2):
            wr[p][0].wait()
            wr[p][1].wait()

    return k(x, dests3)


def _sc_gather(out_all, destg3):
    mesh = plsc.VectorSubcoreMesh(core_axis_name="c", subcore_axis_name="s")

    @functools.partial(
        pl.kernel, mesh=mesh,
        out_type=jax.ShapeDtypeStruct((N_PAIR, DIM), jnp.float32),
        scratch_types=[
            pltpu.VMEM((CHUNKS_PER_W, CHUNK), jnp.int32),
            pltpu.VMEM((2, CHUNK, DIM), jnp.float32),
            pltpu.SemaphoreType.DMA,
            pltpu.SemaphoreType.DMA,
            pltpu.SemaphoreType.DMA,
            pltpu.SemaphoreType.DMA,
        ],
    )
    def k(src_hbm, d_hbm, yg_hbm, idx_v, rows_v, g0, g1, w0, w1):
        wid = lax.axis_index("s") * NC + lax.axis_index("c")
        pltpu.sync_copy(d_hbm.at[wid], idx_v)
        gsem = [g0, g1]
        wsem = [w0, w1]
        gd = [None, None]
        wr = [None, None]
        base = wid * PAIR_PER_W
        for c in range(CHUNKS_PER_W):
            p = c % 2
            if c >= 2:
                wr[p].wait()
            gd[p] = pltpu.async_copy(src_hbm.at[idx_v.at[c]],
                                     rows_v.at[p], gsem[p])
            if c >= 1:
                q = (c - 1) % 2
                gd[q].wait()
                wr[q] = pltpu.async_copy(
                    rows_v.at[q],
                    yg_hbm.at[pl.ds(base + (c - 1) * CHUNK, CHUNK)], wsem[q])
        q = (CHUNKS_PER_W - 1) % 2
        gd[q].wait()
        wr[q] = pltpu.async_copy(
            rows_v.at[q],
            yg_hbm.at[pl.ds(base + (CHUNKS_PER_W - 1) * CHUNK, CHUNK)], wsem[q])
        wr[0].wait()
        wr[1].wait()

    return k(out_all, destg3)


def _ffn_body(xg_ref, w1_ref, w3_ref, w2_ref, sv_ref, out_ref):
    xb = xg_ref[...]
    a = lax.dot_general(xb, w1_ref[0], (((1,), (1,)), ((), ())),
                        preferred_element_type=jnp.float32)
    g = lax.dot_general(xb, w3_ref[0], (((1,), (1,)), ((), ())),
                        preferred_element_type=jnp.float32)
    h = a * jax.nn.sigmoid(a) * g
    out = lax.dot_general(h, w2_ref[0], (((1,), (1,)), ((), ())),
                          preferred_element_type=jnp.float32)
    sv = sv_ref[0]
    out_ref[...] = jnp.where(sv > 0.5, out, 0.0)


def _ffn(xg, w1, w2, w3, slotvalid):
    return pl.pallas_call(
        _ffn_body,
        grid=(E,),
        in_specs=[
            pl.BlockSpec((CAP, DIM), lambda e: (e, 0)),
            pl.BlockSpec((1, INTER, DIM), lambda e: (e, 0, 0)),
            pl.BlockSpec((1, INTER, DIM), lambda e: (e, 0, 0)),
            pl.BlockSpec((1, DIM, INTER), lambda e: (e, 0, 0)),
            pl.BlockSpec((1, CAP, 1), lambda e: (e, 0, 0)),
        ],
        out_specs=pl.BlockSpec((CAP, DIM), lambda e: (e, 0)),
        out_shape=jax.ShapeDtypeStruct((E * CAP, DIM), jnp.float32),
    )(xg[:E * CAP], w1, w3, w2, slotvalid)


def _shared_body(x_ref, sw1_ref, sw3_ref, sw2_ref, z_ref):
    xb = x_ref[...]
    a = lax.dot_general(xb, sw1_ref[...], (((1,), (1,)), ((), ())),
                        preferred_element_type=jnp.float32)
    g = lax.dot_general(xb, sw3_ref[...], (((1,), (1,)), ((), ())),
                        preferred_element_type=jnp.float32)
    h = a * jax.nn.sigmoid(a) * g
    z_ref[...] = lax.dot_general(h, sw2_ref[...], (((1,), (1,)), ((), ())),
                                 preferred_element_type=jnp.float32)


def _shared(x, sw1, sw2, sw3):
    return pl.pallas_call(
        _shared_body,
        grid=(NB,),
        in_specs=[
            pl.BlockSpec((TB, DIM), lambda b: (b, 0)),
            pl.BlockSpec((DIM, DIM), lambda b: (0, 0)),
            pl.BlockSpec((DIM, DIM), lambda b: (0, 0)),
            pl.BlockSpec((DIM, DIM), lambda b: (0, 0)),
        ],
        out_specs=pl.BlockSpec((TB, DIM), lambda b: (b, 0)),
        out_shape=jax.ShapeDtypeStruct((N_TOK, DIM), jnp.float32),
    )(x, sw1, sw3, sw2)


def _combine_body(z_ref, yg_ref, f_ref, y_ref):
    f0 = f_ref[:, 0:1]
    f1 = f_ref[:, 1:2]
    y_ref[...] = z_ref[...] + f0 * yg_ref[0, 0] + f1 * yg_ref[0, 1]


def _combine(z, yg, f):
    return pl.pallas_call(
        _combine_body,
        grid=(NB,),
        in_specs=[
            pl.BlockSpec((TB, DIM), lambda b: (b, 0)),
            pl.BlockSpec((1, 2, TB, DIM), lambda b: (b, 0, 0, 0)),
            pl.BlockSpec((TB, 2), lambda b: (b, 0)),
        ],
        out_specs=pl.BlockSpec((TB, DIM), lambda b: (b, 0)),
        out_shape=jax.ShapeDtypeStruct((N_TOK, DIM), jnp.float32),
    )(z, yg, f)


def kernel(x, gate_w, w1, w2, w3, sw1, sw2, sw3):
    dest_s, dest_g, f, slotvalid = _router(x, gate_w)
    dests3 = dest_s.reshape(NW, 2 * 4, CHUNK)
    destg3 = dest_g.reshape(NW, CHUNKS_PER_W, CHUNK)
    xg = _sc_dispatch(x, dests3)
    out_all = _ffn(xg, w1, w2, w3, slotvalid)
    yg = _sc_gather(out_all, destg3).reshape(NB, 2, TB, DIM)
    z = _shared(x, sw1, sw2, sw3)
    return _combine(z, yg, f)


# speedup vs baseline: 1.0535x; 1.0070x over previous
import functools

import jax
import jax.numpy as jnp
from jax import lax
from jax.experimental import pallas as pl
from jax.experimental.pallas import tpu as pltpu
from jax.experimental.pallas import tpu_sc as plsc

DIM = 768
INTER = 384
E = 64
CAP = 160
N_TOK = 4096
NROWS = E * CAP + CAP
N_PAIR = 2 * N_TOK

TB = 256
NB = N_TOK // TB

NC = 2
NS = 16
NW = NC * NS
CHUNK = 32
PAIR_PER_W = N_PAIR // NW
CHUNKS_PER_W = PAIR_PER_W // CHUNK


def _router_body(x_ref, gw_ref, ds_ref, dg_ref, f_ref, sv_ref, carry_ref):
    b = pl.program_id(0)

    @pl.when(b == 0)
    def _():
        carry_ref[0:1, :] = jnp.zeros((1, E), jnp.float32)

    xb = x_ref[...]
    logits = lax.dot_general(xb, gw_ref[...],
                             (((1,), (1,)), ((), ())),
                             preferred_element_type=jnp.float32)
    m = jnp.max(logits, axis=1, keepdims=True)
    p = jnp.exp(logits - m)
    scores = p / jnp.sum(p, axis=1, keepdims=True)

    lane = lax.broadcasted_iota(jnp.int32, (TB, E), 1)
    s1 = jnp.max(scores, axis=1, keepdims=True)
    e0 = jnp.min(jnp.where(scores == s1, lane, E), axis=1, keepdims=True)
    sc2 = jnp.where(lane == e0, -1.0, scores)
    s2 = jnp.max(sc2, axis=1, keepdims=True)
    e1 = jnp.min(jnp.where(sc2 == s2, lane, E), axis=1, keepdims=True)
    denom = s1 + s2 + 1e-9
    w0 = s1 / denom
    w1 = s2 / denom

    oh0 = (lane == e0).astype(jnp.float32)
    oh1 = (lane == e1).astype(jnp.float32)
    cb = oh0 + oh1

    r = lax.broadcasted_iota(jnp.int32, (TB, TB), 0)
    c = lax.broadcasted_iota(jnp.int32, (TB, TB), 1)
    tril = (r > c).astype(jnp.float32)
    carry = carry_ref[0:1, :]
    base = lax.dot_general(tril, cb, (((1,), (0,)), ((), ())),
                           preferred_element_type=jnp.float32) + carry
    pos0 = jnp.sum(base * oh0, axis=1, keepdims=True).astype(jnp.int32)
    pos1 = jnp.sum(base * oh1, axis=1, keepdims=True).astype(jnp.int32)
    new_carry = carry + jnp.sum(cb, axis=0, keepdims=True)
    carry_ref[0:1, :] = new_carry

    v0 = pos0 < CAP
    v1 = pos1 < CAP
    d0 = e0 * CAP + pos0
    d1 = e1 * CAP + pos1
    tglob = b * TB + lax.broadcasted_iota(jnp.int32, (TB, 1), 0)
    dump = E * CAP + lax.rem(tglob, 32)
    s0 = jnp.where(v0, d0, dump)
    s1 = jnp.where(v1, d1, dump)
    ds_ref[...] = jnp.concatenate(
        [s0.reshape(1, 2, 4, 1, CHUNK), s1.reshape(1, 2, 4, 1, CHUNK)],
        axis=3).reshape(1, 2, 8, CHUNK)
    dg_ref[...] = jnp.concatenate(
        [jnp.where(v0, d0, 0).reshape(1, 1, TB),
         jnp.where(v1, d1, 0).reshape(1, 1, TB)], axis=1)
    f_ref[...] = jnp.concatenate(
        [jnp.where(v0, w0, 0.0), jnp.where(v1, w1, 0.0)], axis=1)

    @pl.when(b == NB - 1)
    def _():
        counts = new_carry.astype(jnp.int32).reshape(E, 1, 1)
        sl = lax.broadcasted_iota(jnp.int32, (E, CAP, 1), 1)
        sv_ref[...] = (sl < counts).astype(jnp.float32)


def _router(x, gate_w):
    return pl.pallas_call(
        _router_body,
        grid=(NB,),
        in_specs=[
            pl.BlockSpec((TB, DIM), lambda b: (b, 0)),
            pl.BlockSpec((E, DIM), lambda b: (0, 0)),
        ],
        out_specs=[
            pl.BlockSpec((1, 2, 8, CHUNK), lambda b: (b, 0, 0, 0)),
            pl.BlockSpec((1, 2, TB), lambda b: (b, 0, 0)),
            pl.BlockSpec((TB, 2), lambda b: (b, 0)),
            pl.BlockSpec((E, CAP, 1), lambda b: (0, 0, 0)),
        ],
        out_shape=[
            jax.ShapeDtypeStruct((NB, 2, 8, CHUNK), jnp.int32),
            jax.ShapeDtypeStruct((NB, 2, TB), jnp.int32),
            jax.ShapeDtypeStruct((N_TOK, 2), jnp.float32),
            jax.ShapeDtypeStruct((E, CAP, 1), jnp.float32),
        ],
        scratch_shapes=[pltpu.VMEM((8, E), jnp.float32)],
    )(x, gate_w)


def _sc_dispatch(x, dests3):
    mesh = plsc.VectorSubcoreMesh(core_axis_name="c", subcore_axis_name="s")

    nck = 4

    @functools.partial(
        pl.kernel, mesh=mesh,
        out_type=jax.ShapeDtypeStruct((NROWS, DIM), jnp.float32),
        scratch_types=[
            pltpu.VMEM((2 * nck, CHUNK), jnp.int32),
            pltpu.VMEM((2, CHUNK, DIM), jnp.float32),
            pltpu.SemaphoreType.DMA,
            pltpu.SemaphoreType.DMA,
            pltpu.SemaphoreType.DMA,
            pltpu.SemaphoreType.DMA,
        ],
    )
    def k(x_hbm, d_hbm, xg_hbm, idx_v, rows_v, r0, r1, s0, s1):
        wid = lax.axis_index("s") * NC + lax.axis_index("c")
        base_tok = wid * (nck * CHUNK)
        pltpu.sync_copy(d_hbm.at[wid], idx_v)
        rsem = [r0, r1]
        ssem = [s0, s1]
        rd = [None, None]
        wr = [None, None]
        for c in range(nck):
            p = c % 2
            if c >= 2:
                wr[p][0].wait()
                wr[p][1].wait()
            rd[p] = pltpu.async_copy(
                x_hbm.at[pl.ds(base_tok + c * CHUNK, CHUNK)],
                rows_v.at[p], rsem[p])
            if c >= 1:
                q = (c - 1) % 2
                rd[q].wait()
                wr[q] = (
                    pltpu.async_copy(rows_v.at[q],
                                     xg_hbm.at[idx_v.at[2 * (c - 1)]], ssem[q]),
                    pltpu.async_copy(rows_v.at[q],
                                     xg_hbm.at[idx_v.at[2 * (c - 1) + 1]],
                                     ssem[q]),
                )
        q = (nck - 1) % 2
        rd[q].wait()
        wr[q] = (
            pltpu.async_copy(rows_v.at[q],
                             xg_hbm.at[idx_v.at[2 * (nck - 1)]], ssem[q]),
            pltpu.async_copy(rows_v.at[q],
                             xg_hbm.at[idx_v.at[2 * (nck - 1) + 1]], ssem[q]),
        )
        for p in range(2):
            wr[p][0].wait()
            wr[p][1].wait()

    return k(x, dests3)


def _sc_gather(out_all, destg3):
    mesh = plsc.VectorSubcoreMesh(core_axis_name="c", subcore_axis_name="s")

    @functools.partial(
        pl.kernel, mesh=mesh,
        out_type=jax.ShapeDtypeStruct((N_PAIR, DIM), jnp.float32),
        scratch_types=[
            pltpu.VMEM((CHUNKS_PER_W, CHUNK), jnp.int32),
            pltpu.VMEM((2, CHUNK, DIM), jnp.float32),
            pltpu.SemaphoreType.DMA,
            pltpu.SemaphoreType.DMA,
            pltpu.SemaphoreType.DMA,
            pltpu.SemaphoreType.DMA,
        ],
    )
    def k(src_hbm, d_hbm, yg_hbm, idx_v, rows_v, g0, g1, w0, w1):
        wid = lax.axis_index("s") * NC + lax.axis_index("c")
        pltpu.sync_copy(d_hbm.at[wid], idx_v)
        gsem = [g0, g1]
        wsem = [w0, w1]
        gd = [None, None]
        wr = [None, None]
        base = wid * PAIR_PER_W
        for c in range(CHUNKS_PER_W):
            p = c % 2
            if c >= 2:
                wr[p].wait()
            gd[p] = pltpu.async_copy(src_hbm.at[idx_v.at[c]],
                                     rows_v.at[p], gsem[p])
            if c >= 1:
                q = (c - 1) % 2
                gd[q].wait()
                wr[q] = pltpu.async_copy(
                    rows_v.at[q],
                    yg_hbm.at[pl.ds(base + (c - 1) * CHUNK, CHUNK)], wsem[q])
        q = (CHUNKS_PER_W - 1) % 2
        gd[q].wait()
        wr[q] = pltpu.async_copy(
            rows_v.at[q],
            yg_hbm.at[pl.ds(base + (CHUNKS_PER_W - 1) * CHUNK, CHUNK)], wsem[q])
        wr[0].wait()
        wr[1].wait()

    return k(out_all, destg3)


def _ffn_body(xg_ref, w1_ref, w3_ref, w2_ref, sv_ref, out_ref):
    xb = xg_ref[...]
    a = lax.dot_general(xb, w1_ref[0], (((1,), (1,)), ((), ())),
                        preferred_element_type=jnp.float32)
    g = lax.dot_general(xb, w3_ref[0], (((1,), (1,)), ((), ())),
                        preferred_element_type=jnp.float32)
    h = a * jax.nn.sigmoid(a) * g
    out = lax.dot_general(h, w2_ref[0], (((1,), (1,)), ((), ())),
                          preferred_element_type=jnp.float32)
    sv = sv_ref[0]
    out_ref[...] = jnp.where(sv > 0.5, out, 0.0)


def _ffn(xg, w1, w2, w3, slotvalid):
    return pl.pallas_call(
        _ffn_body,
        grid=(E,),
        in_specs=[
            pl.BlockSpec((CAP, DIM), lambda e: (e, 0)),
            pl.BlockSpec((1, INTER, DIM), lambda e: (e, 0, 0)),
            pl.BlockSpec((1, INTER, DIM), lambda e: (e, 0, 0)),
            pl.BlockSpec((1, DIM, INTER), lambda e: (e, 0, 0)),
            pl.BlockSpec((1, CAP, 1), lambda e: (e, 0, 0)),
        ],
        out_specs=pl.BlockSpec((CAP, DIM), lambda e: (e, 0)),
        out_shape=jax.ShapeDtypeStruct((E * CAP, DIM), jnp.float32),
    )(xg[:E * CAP], w1, w3, w2, slotvalid)


def _final_body(x_ref, sw1_ref, sw3_ref, sw2_ref, yg_ref, f_ref, y_ref):
    xb = x_ref[...]
    a = lax.dot_general(xb, sw1_ref[...], (((1,), (1,)), ((), ())),
                        preferred_element_type=jnp.float32)
    g = lax.dot_general(xb, sw3_ref[...], (((1,), (1,)), ((), ())),
                        preferred_element_type=jnp.float32)
    h = a * jax.nn.sigmoid(a) * g
    z = lax.dot_general(h, sw2_ref[...], (((1,), (1,)), ((), ())),
                        preferred_element_type=jnp.float32)
    f0 = f_ref[:, 0:1]
    f1 = f_ref[:, 1:2]
    y_ref[...] = z + f0 * yg_ref[0, 0] + f1 * yg_ref[0, 1]


def _final(x, sw1, sw2, sw3, yg, f):
    return pl.pallas_call(
        _final_body,
        grid=(NB,),
        in_specs=[
            pl.BlockSpec((TB, DIM), lambda b: (b, 0)),
            pl.BlockSpec((DIM, DIM), lambda b: (0, 0)),
            pl.BlockSpec((DIM, DIM), lambda b: (0, 0)),
            pl.BlockSpec((DIM, DIM), lambda b: (0, 0)),
            pl.BlockSpec((1, 2, TB, DIM), lambda b: (b, 0, 0, 0)),
            pl.BlockSpec((TB, 2), lambda b: (b, 0)),
        ],
        out_specs=pl.BlockSpec((TB, DIM), lambda b: (b, 0)),
        out_shape=jax.ShapeDtypeStruct((N_TOK, DIM), jnp.float32),
    )(x, sw1, sw3, sw2, yg, f)


def kernel(x, gate_w, w1, w2, w3, sw1, sw2, sw3):
    dest_s, dest_g, f, slotvalid = _router(x, gate_w)
    dests3 = dest_s.reshape(NW, 2 * 4, CHUNK)
    destg3 = dest_g.reshape(NW, CHUNKS_PER_W, CHUNK)
    xg = _sc_dispatch(x, dests3)
    out_all = _ffn(xg, w1, w2, w3, slotvalid)
    yg = _sc_gather(out_all, destg3).reshape(NB, 2, TB, DIM)
    return _final(x, sw1, sw2, sw3, yg, f)


# speedup vs baseline: 1.0612x; 1.0074x over previous
import functools

import jax
import jax.numpy as jnp
from jax import lax
from jax.experimental import pallas as pl
from jax.experimental.pallas import tpu as pltpu
from jax.experimental.pallas import tpu_sc as plsc

DIM = 768
INTER = 384
E = 64
CAP = 160
N_TOK = 4096
NROWS = E * CAP + CAP
N_PAIR = 2 * N_TOK

TB = 256
NB = N_TOK // TB

NC = 2
NS = 16
NW = NC * NS
CHUNK = 32
PAIR_PER_W = N_PAIR // NW
CHUNKS_PER_W = PAIR_PER_W // CHUNK


def _router_body(x_ref, gw_ref, ds_ref, dg_ref, f_ref, sv_ref, carry_ref):
    b = pl.program_id(0)

    @pl.when(b == 0)
    def _():
        carry_ref[0:1, :] = jnp.zeros((1, E), jnp.float32)

    xb = x_ref[...]
    logits = lax.dot_general(xb, gw_ref[...],
                             (((1,), (1,)), ((), ())),
                             preferred_element_type=jnp.float32)
    lane = lax.broadcasted_iota(jnp.int32, (TB, E), 1)
    m1 = jnp.max(logits, axis=1, keepdims=True)
    e0 = jnp.min(jnp.where(logits == m1, lane, E), axis=1, keepdims=True)
    sc2 = jnp.where(lane == e0, -jnp.inf, logits)
    m2 = jnp.max(sc2, axis=1, keepdims=True)
    e1 = jnp.min(jnp.where(sc2 == m2, lane, E), axis=1, keepdims=True)
    w0 = jax.nn.sigmoid(m1 - m2)
    w1 = jax.nn.sigmoid(m2 - m1)

    oh0 = (lane == e0).astype(jnp.float32)
    oh1 = (lane == e1).astype(jnp.float32)
    cb = oh0 + oh1

    r = lax.broadcasted_iota(jnp.int32, (TB, TB), 0)
    c = lax.broadcasted_iota(jnp.int32, (TB, TB), 1)
    tril = (r > c).astype(jnp.float32)
    carry = carry_ref[0:1, :]
    base = lax.dot_general(tril, cb, (((1,), (0,)), ((), ())),
                           preferred_element_type=jnp.float32) + carry
    pos0 = jnp.sum(base * oh0, axis=1, keepdims=True).astype(jnp.int32)
    pos1 = jnp.sum(base * oh1, axis=1, keepdims=True).astype(jnp.int32)
    new_carry = carry + jnp.sum(cb, axis=0, keepdims=True)
    carry_ref[0:1, :] = new_carry

    v0 = pos0 < CAP
    v1 = pos1 < CAP
    d0 = e0 * CAP + pos0
    d1 = e1 * CAP + pos1
    tglob = b * TB + lax.broadcasted_iota(jnp.int32, (TB, 1), 0)
    dump = E * CAP + lax.rem(tglob, 32)
    s0 = jnp.where(v0, d0, dump)
    s1 = jnp.where(v1, d1, dump)
    ds_ref[...] = jnp.concatenate(
        [s0.reshape(1, 2, 4, 1, CHUNK), s1.reshape(1, 2, 4, 1, CHUNK)],
        axis=3).reshape(1, 2, 8, CHUNK)
    dg_ref[...] = jnp.concatenate(
        [jnp.where(v0, d0, 0).reshape(1, 1, TB),
         jnp.where(v1, d1, 0).reshape(1, 1, TB)], axis=1)
    f_ref[...] = jnp.concatenate(
        [jnp.where(v0, w0, 0.0), jnp.where(v1, w1, 0.0)], axis=1)

    @pl.when(b == NB - 1)
    def _():
        counts = new_carry.astype(jnp.int32).reshape(E, 1, 1)
        sl = lax.broadcasted_iota(jnp.int32, (E, CAP, 1), 1)
        sv_ref[...] = (sl < counts).astype(jnp.float32)


def _router(x, gate_w):
    return pl.pallas_call(
        _router_body,
        grid=(NB,),
        in_specs=[
            pl.BlockSpec((TB, DIM), lambda b: (b, 0)),
            pl.BlockSpec((E, DIM), lambda b: (0, 0)),
        ],
        out_specs=[
            pl.BlockSpec((1, 2, 8, CHUNK), lambda b: (b, 0, 0, 0)),
            pl.BlockSpec((1, 2, TB), lambda b: (b, 0, 0)),
            pl.BlockSpec((TB, 2), lambda b: (b, 0)),
            pl.BlockSpec((E, CAP, 1), lambda b: (0, 0, 0)),
        ],
        out_shape=[
            jax.ShapeDtypeStruct((NB, 2, 8, CHUNK), jnp.int32),
            jax.ShapeDtypeStruct((NB, 2, TB), jnp.int32),
            jax.ShapeDtypeStruct((N_TOK, 2), jnp.float32),
            jax.ShapeDtypeStruct((E, CAP, 1), jnp.float32),
        ],
        scratch_shapes=[pltpu.VMEM((8, E), jnp.float32)],
    )(x, gate_w)


def _sc_dispatch(x, dests3):
    mesh = plsc.VectorSubcoreMesh(core_axis_name="c", subcore_axis_name="s")

    nck = 4

    @functools.partial(
        pl.kernel, mesh=mesh,
        out_type=jax.ShapeDtypeStruct((NROWS, DIM), jnp.float32),
        scratch_types=[
            pltpu.VMEM((2 * nck, CHUNK), jnp.int32),
            pltpu.VMEM((2, CHUNK, DIM), jnp.float32),
            pltpu.SemaphoreType.DMA,
            pltpu.SemaphoreType.DMA,
            pltpu.SemaphoreType.DMA,
            pltpu.SemaphoreType.DMA,
        ],
    )
    def k(x_hbm, d_hbm, xg_hbm, idx_v, rows_v, r0, r1, s0, s1):
        wid = lax.axis_index("s") * NC + lax.axis_index("c")
        base_tok = wid * (nck * CHUNK)
        pltpu.sync_copy(d_hbm.at[wid], idx_v)
        rsem = [r0, r1]
        ssem = [s0, s1]
        rd = [None, None]
        wr = [None, None]
        for c in range(nck):
            p = c % 2
            if c >= 2:
                wr[p][0].wait()
                wr[p][1].wait()
            rd[p] = pltpu.async_copy(
                x_hbm.at[pl.ds(base_tok + c * CHUNK, CHUNK)],
                rows_v.at[p], rsem[p])
            if c >= 1:
                q = (c - 1) % 2
                rd[q].wait()
                wr[q] = (
                    pltpu.async_copy(rows_v.at[q],
                                     xg_hbm.at[idx_v.at[2 * (c - 1)]], ssem[q]),
                    pltpu.async_copy(rows_v.at[q],
                                     xg_hbm.at[idx_v.at[2 * (c - 1) + 1]],
                                     ssem[q]),
                )
        q = (nck - 1) % 2
        rd[q].wait()
        wr[q] = (
            pltpu.async_copy(rows_v.at[q],
                             xg_hbm.at[idx_v.at[2 * (nck - 1)]], ssem[q]),
            pltpu.async_copy(rows_v.at[q],
                             xg_hbm.at[idx_v.at[2 * (nck - 1) + 1]], ssem[q]),
        )
        for p in range(2):
            wr[p][0].wait()
            wr[p][1].wait()

    return k(x, dests3)


def _sc_gather(out_all, destg3):
    mesh = plsc.VectorSubcoreMesh(core_axis_name="c", subcore_axis_name="s")

    @functools.partial(
        pl.kernel, mesh=mesh,
        out_type=jax.ShapeDtypeStruct((N_PAIR, DIM), jnp.float32),
        scratch_types=[
            pltpu.VMEM((CHUNKS_PER_W, CHUNK), jnp.int32),
            pltpu.VMEM((2, CHUNK, DIM), jnp.float32),
            pltpu.SemaphoreType.DMA,
            pltpu.SemaphoreType.DMA,
            pltpu.SemaphoreType.DMA,
            pltpu.SemaphoreType.DMA,
        ],
    )
    def k(src_hbm, d_hbm, yg_hbm, idx_v, rows_v, g0, g1, w0, w1):
        wid = lax.axis_index("s") * NC + lax.axis_index("c")
        pltpu.sync_copy(d_hbm.at[wid], idx_v)
        gsem = [g0, g1]
        wsem = [w0, w1]
        gd = [None, None]
        wr = [None, None]
        base = wid * PAIR_PER_W
        for c in range(CHUNKS_PER_W):
            p = c % 2
            if c >= 2:
                wr[p].wait()
            gd[p] = pltpu.async_copy(src_hbm.at[idx_v.at[c]],
                                     rows_v.at[p], gsem[p])
            if c >= 1:
                q = (c - 1) % 2
                gd[q].wait()
                wr[q] = pltpu.async_copy(
                    rows_v.at[q],
                    yg_hbm.at[pl.ds(base + (c - 1) * CHUNK, CHUNK)], wsem[q])
        q = (CHUNKS_PER_W - 1) % 2
        gd[q].wait()
        wr[q] = pltpu.async_copy(
            rows_v.at[q],
            yg_hbm.at[pl.ds(base + (CHUNKS_PER_W - 1) * CHUNK, CHUNK)], wsem[q])
        wr[0].wait()
        wr[1].wait()

    return k(out_all, destg3)


def _ffn_body(xg_ref, w1_ref, w3_ref, w2_ref, sv_ref, out_ref):
    xb = xg_ref[...]
    a = lax.dot_general(xb, w1_ref[0], (((1,), (1,)), ((), ())),
                        preferred_element_type=jnp.float32)
    g = lax.dot_general(xb, w3_ref[0], (((1,), (1,)), ((), ())),
                        preferred_element_type=jnp.float32)
    h = a * jax.nn.sigmoid(a) * g
    out = lax.dot_general(h, w2_ref[0], (((1,), (1,)), ((), ())),
                          preferred_element_type=jnp.float32)
    sv = sv_ref[0]
    out_ref[...] = jnp.where(sv > 0.5, out, 0.0)


def _ffn(xg, w1, w2, w3, slotvalid):
    return pl.pallas_call(
        _ffn_body,
        grid=(E,),
        in_specs=[
            pl.BlockSpec((CAP, DIM), lambda e: (e, 0)),
            pl.BlockSpec((1, INTER, DIM), lambda e: (e, 0, 0)),
            pl.BlockSpec((1, INTER, DIM), lambda e: (e, 0, 0)),
            pl.BlockSpec((1, DIM, INTER), lambda e: (e, 0, 0)),
            pl.BlockSpec((1, CAP, 1), lambda e: (e, 0, 0)),
        ],
        out_specs=pl.BlockSpec((CAP, DIM), lambda e: (e, 0)),
        out_shape=jax.ShapeDtypeStruct((E * CAP, DIM), jnp.float32),
    )(xg[:E * CAP], w1, w3, w2, slotvalid)


def _final_body(x_ref, sw1_ref, sw3_ref, sw2_ref, yg_ref, f_ref, y_ref):
    xb = x_ref[...]
    a = lax.dot_general(xb, sw1_ref[...], (((1,), (1,)), ((), ())),
                        preferred_element_type=jnp.float32)
    g = lax.dot_general(xb, sw3_ref[...], (((1,), (1,)), ((), ())),
                        preferred_element_type=jnp.float32)
    h = a * jax.nn.sigmoid(a) * g
    z = lax.dot_general(h, sw2_ref[...], (((1,), (1,)), ((), ())),
                        preferred_element_type=jnp.float32)
    f0 = f_ref[:, 0:1]
    f1 = f_ref[:, 1:2]
    y_ref[...] = z + f0 * yg_ref[0, 0] + f1 * yg_ref[0, 1]


def _final(x, sw1, sw2, sw3, yg, f):
    return pl.pallas_call(
        _final_body,
        grid=(NB,),
        in_specs=[
            pl.BlockSpec((TB, DIM), lambda b: (b, 0)),
            pl.BlockSpec((DIM, DIM), lambda b: (0, 0)),
            pl.BlockSpec((DIM, DIM), lambda b: (0, 0)),
            pl.BlockSpec((DIM, DIM), lambda b: (0, 0)),
            pl.BlockSpec((1, 2, TB, DIM), lambda b: (b, 0, 0, 0)),
            pl.BlockSpec((TB, 2), lambda b: (b, 0)),
        ],
        out_specs=pl.BlockSpec((TB, DIM), lambda b: (b, 0)),
        out_shape=jax.ShapeDtypeStruct((N_TOK, DIM), jnp.float32),
    )(x, sw1, sw3, sw2, yg, f)


def kernel(x, gate_w, w1, w2, w3, sw1, sw2, sw3):
    dest_s, dest_g, f, slotvalid = _router(x, gate_w)
    dests3 = dest_s.reshape(NW, 2 * 4, CHUNK)
    destg3 = dest_g.reshape(NW, CHUNKS_PER_W, CHUNK)
    xg = _sc_dispatch(x, dests3)
    out_all = _ffn(xg, w1, w2, w3, slotvalid)
    yg = _sc_gather(out_all, destg3).reshape(NB, 2, TB, DIM)
    return _final(x, sw1, sw2, sw3, yg, f)


# speedup vs baseline: 1.1623x; 1.0952x over previous
import functools

import jax
import jax.numpy as jnp
from jax import lax
from jax.experimental import pallas as pl
from jax.experimental.pallas import tpu as pltpu
from jax.experimental.pallas import tpu_sc as plsc

DIM = 768
INTER = 384
E = 64
CAP = 160
N_TOK = 4096
NROWS = E * CAP + CAP
N_PAIR = 2 * N_TOK

TB = 256
NB = N_TOK // TB

NC = 2
NS = 16
NW = NC * NS
CHUNK = 32
PAIR_PER_W = N_PAIR // NW
CHUNKS_PER_W = PAIR_PER_W // CHUNK


def _router_body(x_ref, gw_ref, ds_ref, dg_ref, f_ref, sv_ref, carry_ref):
    b = pl.program_id(0)

    @pl.when(b == 0)
    def _():
        carry_ref[0:1, :] = jnp.zeros((1, E), jnp.float32)

    xb = x_ref[...]
    logits = lax.dot_general(xb, gw_ref[...],
                             (((1,), (1,)), ((), ())),
                             preferred_element_type=jnp.float32)
    lane = lax.broadcasted_iota(jnp.int32, (TB, E), 1)
    m1 = jnp.max(logits, axis=1, keepdims=True)
    e0 = jnp.min(jnp.where(logits == m1, lane, E), axis=1, keepdims=True)
    sc2 = jnp.where(lane == e0, -jnp.inf, logits)
    m2 = jnp.max(sc2, axis=1, keepdims=True)
    e1 = jnp.min(jnp.where(sc2 == m2, lane, E), axis=1, keepdims=True)
    w0 = jax.nn.sigmoid(m1 - m2)
    w1 = jax.nn.sigmoid(m2 - m1)

    oh0 = (lane == e0).astype(jnp.float32)
    oh1 = (lane == e1).astype(jnp.float32)
    cb = oh0 + oh1

    r = lax.broadcasted_iota(jnp.int32, (TB, TB), 0)
    c = lax.broadcasted_iota(jnp.int32, (TB, TB), 1)
    tril = (r > c).astype(jnp.float32)
    carry = carry_ref[0:1, :]
    base = lax.dot_general(tril, cb, (((1,), (0,)), ((), ())),
                           preferred_element_type=jnp.float32) + carry
    pos0 = jnp.sum(base * oh0, axis=1, keepdims=True).astype(jnp.int32)
    pos1 = jnp.sum(base * oh1, axis=1, keepdims=True).astype(jnp.int32)
    new_carry = carry + jnp.sum(cb, axis=0, keepdims=True)
    carry_ref[0:1, :] = new_carry

    v0 = pos0 < CAP
    v1 = pos1 < CAP
    d0 = e0 * CAP + pos0
    d1 = e1 * CAP + pos1
    tglob = b * TB + lax.broadcasted_iota(jnp.int32, (TB, 1), 0)
    dump = E * CAP + lax.rem(tglob, 32)
    s0 = jnp.where(v0, d0, dump)
    s1 = jnp.where(v1, d1, dump)
    ds_ref[...] = jnp.concatenate(
        [s0.reshape(1, 2, 4, 1, CHUNK), s1.reshape(1, 2, 4, 1, CHUNK)],
        axis=3).reshape(1, 2, 8, CHUNK)
    dg_ref[...] = jnp.concatenate(
        [jnp.where(v0, d0, 0).reshape(1, 1, TB),
         jnp.where(v1, d1, 0).reshape(1, 1, TB)], axis=1)
    f_ref[...] = jnp.concatenate(
        [jnp.where(v0, w0, 0.0), jnp.where(v1, w1, 0.0)], axis=1)

    @pl.when(b == NB - 1)
    def _():
        counts = new_carry.astype(jnp.int32).reshape(E, 1, 1)
        sl = lax.broadcasted_iota(jnp.int32, (E, CAP, 1), 1)
        sv_ref[...] = (sl < counts).astype(jnp.float32)


def _router(x, gate_w):
    return pl.pallas_call(
        _router_body,
        grid=(NB,),
        in_specs=[
            pl.BlockSpec((TB, DIM), lambda b: (b, 0)),
            pl.BlockSpec((E, DIM), lambda b: (0, 0)),
        ],
        out_specs=[
            pl.BlockSpec((1, 2, 8, CHUNK), lambda b: (b, 0, 0, 0)),
            pl.BlockSpec((1, 2, TB), lambda b: (b, 0, 0)),
            pl.BlockSpec((TB, 2), lambda b: (b, 0)),
            pl.BlockSpec((E, CAP, 1), lambda b: (0, 0, 0)),
        ],
        out_shape=[
            jax.ShapeDtypeStruct((NB, 2, 8, CHUNK), jnp.int32),
            jax.ShapeDtypeStruct((NB, 2, TB), jnp.int32),
            jax.ShapeDtypeStruct((N_TOK, 2), jnp.float32),
            jax.ShapeDtypeStruct((E, CAP, 1), jnp.float32),
        ],
        scratch_shapes=[pltpu.VMEM((8, E), jnp.float32)],
    )(x, gate_w)


def _sc_dispatch(x, dests3):
    mesh = plsc.VectorSubcoreMesh(core_axis_name="c", subcore_axis_name="s")

    nck = 4

    @functools.partial(
        pl.kernel, mesh=mesh,
        out_type=jax.ShapeDtypeStruct((NROWS, DIM), jnp.float32),
        scratch_types=[
            pltpu.VMEM((2 * nck, CHUNK), jnp.int32),
            pltpu.VMEM((2, CHUNK, DIM), jnp.float32),
            pltpu.SemaphoreType.DMA,
            pltpu.SemaphoreType.DMA,
            pltpu.SemaphoreType.DMA,
            pltpu.SemaphoreType.DMA,
        ],
    )
    def k(x_hbm, d_hbm, xg_hbm, idx_v, rows_v, r0, r1, s0, s1):
        wid = lax.axis_index("s") * NC + lax.axis_index("c")
        base_tok = wid * (nck * CHUNK)
        pltpu.sync_copy(d_hbm.at[wid], idx_v)
        rsem = [r0, r1]
        ssem = [s0, s1]
        rd = [None, None]
        wr = [None, None]
        for c in range(nck):
            p = c % 2
            if c >= 2:
                wr[p][0].wait()
                wr[p][1].wait()
            rd[p] = pltpu.async_copy(
                x_hbm.at[pl.ds(base_tok + c * CHUNK, CHUNK)],
                rows_v.at[p], rsem[p])
            if c >= 1:
                q = (c - 1) % 2
                rd[q].wait()
                wr[q] = (
                    pltpu.async_copy(rows_v.at[q],
                                     xg_hbm.at[idx_v.at[2 * (c - 1)]], ssem[q]),
                    pltpu.async_copy(rows_v.at[q],
                                     xg_hbm.at[idx_v.at[2 * (c - 1) + 1]],
                                     ssem[q]),
                )
        q = (nck - 1) % 2
        rd[q].wait()
        wr[q] = (
            pltpu.async_copy(rows_v.at[q],
                             xg_hbm.at[idx_v.at[2 * (nck - 1)]], ssem[q]),
            pltpu.async_copy(rows_v.at[q],
                             xg_hbm.at[idx_v.at[2 * (nck - 1) + 1]], ssem[q]),
        )
        for p in range(2):
            wr[p][0].wait()
            wr[p][1].wait()

    return k(x, dests3)


def _sc_gather(out_all, destg3):
    mesh = plsc.VectorSubcoreMesh(core_axis_name="c", subcore_axis_name="s")

    @functools.partial(
        pl.kernel, mesh=mesh,
        out_type=jax.ShapeDtypeStruct((N_PAIR, DIM), jnp.float32),
        scratch_types=[
            pltpu.VMEM((CHUNKS_PER_W, CHUNK), jnp.int32),
            pltpu.VMEM((2, CHUNK, DIM), jnp.float32),
            pltpu.SemaphoreType.DMA,
            pltpu.SemaphoreType.DMA,
            pltpu.SemaphoreType.DMA,
            pltpu.SemaphoreType.DMA,
        ],
    )
    def k(src_hbm, d_hbm, yg_hbm, idx_v, rows_v, g0, g1, w0, w1):
        wid = lax.axis_index("s") * NC + lax.axis_index("c")
        pltpu.sync_copy(d_hbm.at[wid], idx_v)
        gsem = [g0, g1]
        wsem = [w0, w1]
        gd = [None, None]
        wr = [None, None]
        base = wid * PAIR_PER_W
        for c in range(CHUNKS_PER_W):
            p = c % 2
            if c >= 2:
                wr[p].wait()
            gd[p] = pltpu.async_copy(src_hbm.at[idx_v.at[c]],
                                     rows_v.at[p], gsem[p])
            if c >= 1:
                q = (c - 1) % 2
                gd[q].wait()
                wr[q] = pltpu.async_copy(
                    rows_v.at[q],
                    yg_hbm.at[pl.ds(base + (c - 1) * CHUNK, CHUNK)], wsem[q])
        q = (CHUNKS_PER_W - 1) % 2
        gd[q].wait()
        wr[q] = pltpu.async_copy(
            rows_v.at[q],
            yg_hbm.at[pl.ds(base + (CHUNKS_PER_W - 1) * CHUNK, CHUNK)], wsem[q])
        wr[0].wait()
        wr[1].wait()

    return k(out_all, destg3)


def _ffn_body(xg_ref, w1_ref, w3_ref, w2_ref, sv_ref, out_ref):
    xb = xg_ref[...]
    a = lax.dot_general(xb, w1_ref[0], (((1,), (1,)), ((), ())),
                        preferred_element_type=jnp.float32)
    g = lax.dot_general(xb, w3_ref[0], (((1,), (1,)), ((), ())),
                        preferred_element_type=jnp.float32)
    h = a * jax.nn.sigmoid(a) * g
    out = lax.dot_general(h, w2_ref[0], (((1,), (1,)), ((), ())),
                          preferred_element_type=jnp.float32)
    sv = sv_ref[0]
    out_ref[...] = jnp.where(sv > 0.5, out, 0.0)


def _ffn(xg, w1, w2, w3, slotvalid):
    return pl.pallas_call(
        _ffn_body,
        grid=(E,),
        in_specs=[
            pl.BlockSpec((CAP, DIM), lambda e: (e, 0)),
            pl.BlockSpec((1, INTER, DIM), lambda e: (e, 0, 0)),
            pl.BlockSpec((1, INTER, DIM), lambda e: (e, 0, 0)),
            pl.BlockSpec((1, DIM, INTER), lambda e: (e, 0, 0)),
            pl.BlockSpec((1, CAP, 1), lambda e: (e, 0, 0)),
        ],
        out_specs=pl.BlockSpec((CAP, DIM), lambda e: (e, 0)),
        out_shape=jax.ShapeDtypeStruct((E * CAP, DIM), jnp.float32),
    )(xg, w1, w3, w2, slotvalid)


def _final_body(x_ref, sw1_ref, sw3_ref, sw2_ref, yg_ref, f_ref, y_ref):
    xb = x_ref[...]
    a = lax.dot_general(xb, sw1_ref[...], (((1,), (1,)), ((), ())),
                        preferred_element_type=jnp.float32)
    g = lax.dot_general(xb, sw3_ref[...], (((1,), (1,)), ((), ())),
                        preferred_element_type=jnp.float32)
    h = a * jax.nn.sigmoid(a) * g
    z = lax.dot_general(h, sw2_ref[...], (((1,), (1,)), ((), ())),
                        preferred_element_type=jnp.float32)
    f0 = f_ref[:, 0:1]
    f1 = f_ref[:, 1:2]
    y_ref[...] = z + f0 * yg_ref[0, 0] + f1 * yg_ref[0, 1]


def _final(x, sw1, sw2, sw3, yg, f):
    return pl.pallas_call(
        _final_body,
        grid=(NB,),
        in_specs=[
            pl.BlockSpec((TB, DIM), lambda b: (b, 0)),
            pl.BlockSpec((DIM, DIM), lambda b: (0, 0)),
            pl.BlockSpec((DIM, DIM), lambda b: (0, 0)),
            pl.BlockSpec((DIM, DIM), lambda b: (0, 0)),
            pl.BlockSpec((1, 2, TB, DIM), lambda b: (b, 0, 0, 0)),
            pl.BlockSpec((TB, 2), lambda b: (b, 0)),
        ],
        out_specs=pl.BlockSpec((TB, DIM), lambda b: (b, 0)),
        out_shape=jax.ShapeDtypeStruct((N_TOK, DIM), jnp.float32),
    )(x, sw1, sw3, sw2, yg, f)


def kernel(x, gate_w, w1, w2, w3, sw1, sw2, sw3):
    dest_s, dest_g, f, slotvalid = _router(x, gate_w)
    dests3 = dest_s.reshape(NW, 2 * 4, CHUNK)
    destg3 = dest_g.reshape(NW, CHUNKS_PER_W, CHUNK)
    xg = _sc_dispatch(x, dests3)
    out_all = _ffn(xg, w1, w2, w3, slotvalid)
    yg = _sc_gather(out_all, destg3).reshape(NB, 2, TB, DIM)
    return _final(x, sw1, sw2, sw3, yg, f)
